# Initial kernel scaffold; baseline (speedup 1.0000x reference)
import functools

import jax
import jax.numpy as jnp
from jax import lax
from jax.experimental import pallas as pl
from jax.experimental.pallas import tpu as pltpu
from jax.experimental.pallas import tpu_sc as plsc

VOCAB = 100000
EMBED = 64
SEQ = 200
BATCH = 4096

NC, NS = 2, 16
NW = NC * NS
FLAT = BATCH * SEQ
PER_W = FLAT // NW
GROUP = 2 * SEQ
NGROUP = PER_W // GROUP
XFER = 40
NXFER = GROUP // XFER
LANES = 16

_mesh = plsc.VectorSubcoreMesh(
    core_axis_name="c", subcore_axis_name="s", num_cores=NC, num_subcores=NS
)


@functools.partial(
    pl.kernel,
    out_type=jax.ShapeDtypeStruct((FLAT, EMBED), jnp.float32),
    mesh=_mesh,
    scratch_types=[
        pltpu.VMEM((GROUP,), jnp.int32),
        pltpu.VMEM((GROUP, EMBED), jnp.float32),
        pltpu.VMEM((SEQ, EMBED), jnp.float32),
        pltpu.SemaphoreType.DMA,
    ],
)
def _embed_kernel(x_hbm, tab_hbm, pos_hbm, out_hbm, idx_v, rows_v, pos_v, sem):
    wid = lax.axis_index("s") * NC + lax.axis_index("c")
    base = wid * PER_W
    pltpu.sync_copy(pos_hbm, pos_v)

    def group_body(g, carry):
        off = base + g * GROUP
        pltpu.sync_copy(x_hbm.at[pl.ds(off, GROUP)], idx_v)
        descs = [
            pltpu.async_copy(
                tab_hbm.at[idx_v.at[pl.ds(t * XFER, XFER)]],
                rows_v.at[pl.ds(t * XFER, XFER)],
                sem,
            )
            for t in range(NXFER)
        ]
        for d in descs:
            d.wait()

        def add_body(s, inner):
            for c in range(EMBED // LANES):
                p = pos_v[s, pl.ds(c * LANES, LANES)]
                for rep in range(GROUP // SEQ):
                    r = rep * SEQ + s
                    rows_v[r, pl.ds(c * LANES, LANES)] = (
                        rows_v[r, pl.ds(c * LANES, LANES)] + p
                    )
            return inner

        lax.fori_loop(0, SEQ, add_body, 0)
        pltpu.sync_copy(rows_v, out_hbm.at[pl.ds(off, GROUP)])
        return carry

    lax.fori_loop(0, NGROUP, group_body, 0)


def kernel(x, token_embed, pos_embed):
    x_flat = x.reshape(FLAT).astype(jnp.int32)
    pos2d = pos_embed[0, :SEQ, :]
    out = _embed_kernel(x_flat, token_embed, pos2d)
    return out.reshape(BATCH, SEQ, EMBED)


# speedup vs baseline: 3.4773x; 3.4773x over previous
import functools

import jax
import jax.numpy as jnp
from jax import lax
from jax.experimental import pallas as pl
from jax.experimental.pallas import tpu as pltpu
from jax.experimental.pallas import tpu_sc as plsc

VOCAB = 100000
EMBED = 64
SEQ = 200
BATCH = 4096

NC, NS = 2, 16
NW = NC * NS
FLAT = BATCH * SEQ
PER_W = FLAT // NW
GROUP = 2 * SEQ
NGROUP = PER_W // GROUP
XFER = 40
NXFER = GROUP // XFER
LANES = 16

_mesh = plsc.VectorSubcoreMesh(
    core_axis_name="c", subcore_axis_name="s", num_cores=NC, num_subcores=NS
)


@functools.partial(
    pl.kernel,
    out_type=jax.ShapeDtypeStruct((FLAT, EMBED), jnp.float32),
    mesh=_mesh,
    scratch_types=[
        pltpu.VMEM((GROUP,), jnp.int32),
        pltpu.VMEM((GROUP, EMBED), jnp.float32),
        pltpu.VMEM((SEQ, EMBED), jnp.float32),
        pltpu.SemaphoreType.DMA,
    ],
    compiler_params=pltpu.CompilerParams(use_tc_tiling_on_sc=False),
)
def _embed_kernel(x_hbm, tab_hbm, pos_hbm, out_hbm, idx_v, rows_v, pos_v, sem):
    wid = lax.axis_index("s") * NC + lax.axis_index("c")
    base = wid * PER_W
    pltpu.sync_copy(pos_hbm, pos_v)

    def group_body(g, carry):
        off = base + g * GROUP
        pltpu.sync_copy(x_hbm.at[pl.ds(off, GROUP)], idx_v)
        descs = [
            pltpu.async_copy(
                tab_hbm.at[idx_v.at[pl.ds(t * XFER, XFER)]],
                rows_v.at[pl.ds(t * XFER, XFER)],
                sem,
            )
            for t in range(NXFER)
        ]
        for d in descs:
            d.wait()

        def add_body(s, inner):
            for c in range(EMBED // LANES):
                p = pos_v[s, pl.ds(c * LANES, LANES)]
                for rep in range(GROUP // SEQ):
                    r = rep * SEQ + s
                    rows_v[r, pl.ds(c * LANES, LANES)] = (
                        rows_v[r, pl.ds(c * LANES, LANES)] + p
                    )
            return inner

        lax.fori_loop(0, SEQ, add_body, 0)
        pltpu.sync_copy(rows_v, out_hbm.at[pl.ds(off, GROUP)])
        return carry

    lax.fori_loop(0, NGROUP, group_body, 0)


def kernel(x, token_embed, pos_embed):
    x_flat = x.reshape(FLAT).astype(jnp.int32)
    pos2d = pos_embed[0, :SEQ, :]
    out = _embed_kernel(x_flat, token_embed, pos2d)
    return out.reshape(BATCH, SEQ, EMBED)


# speedup vs baseline: 4.1891x; 1.2047x over previous
import functools

import jax
import jax.numpy as jnp
from jax import lax
from jax.experimental import pallas as pl
from jax.experimental.pallas import tpu as pltpu
from jax.experimental.pallas import tpu_sc as plsc

VOCAB = 100000
EMBED = 64
SEQ = 200
BATCH = 4096

NC, NS = 2, 16
NW = NC * NS
FLAT = BATCH * SEQ
PER_W = FLAT // NW
GROUP = 4 * SEQ
NGROUP = PER_W // GROUP
REPS = GROUP // SEQ
XFER = 40
NXFER = GROUP // XFER
LANES = 16

_mesh = plsc.VectorSubcoreMesh(
    core_axis_name="c", subcore_axis_name="s", num_cores=NC, num_subcores=NS
)


@functools.partial(
    pl.kernel,
    out_type=jax.ShapeDtypeStruct((FLAT, EMBED), jnp.float32),
    mesh=_mesh,
    scratch_types=[
        pltpu.VMEM((GROUP,), jnp.int32),
        pltpu.VMEM((GROUP,), jnp.int32),
        pltpu.VMEM((GROUP, EMBED), jnp.float32),
        pltpu.VMEM((GROUP, EMBED), jnp.float32),
        pltpu.VMEM((SEQ, EMBED), jnp.float32),
        pltpu.SemaphoreType.DMA,
        pltpu.SemaphoreType.DMA,
        pltpu.SemaphoreType.DMA,
        pltpu.SemaphoreType.DMA,
    ],
    compiler_params=pltpu.CompilerParams(use_tc_tiling_on_sc=False),
)
def _embed_kernel(
    x_hbm, tab_hbm, pos_hbm, out_hbm,
    idx0, idx1, rows0, rows1, pos_v, gsem0, gsem1, wsem0, wsem1,
):
    wid = lax.axis_index("s") * NC + lax.axis_index("c")
    base = wid * PER_W
    pltpu.sync_copy(pos_hbm, pos_v)
    bufs = ((idx0, rows0, gsem0, wsem0), (idx1, rows1, gsem1, wsem1))

    def issue(gg, buf):
        idx_v, rows_v, gsem, _ = buf
        off = base + gg * GROUP
        pltpu.sync_copy(x_hbm.at[pl.ds(off, GROUP)], idx_v)
        for t in range(NXFER):
            pltpu.async_copy(
                tab_hbm.at[idx_v.at[pl.ds(t * XFER, XFER)]],
                rows_v.at[pl.ds(t * XFER, XFER)],
                gsem,
            )

    def wait_gathers(buf):
        idx_v, rows_v, gsem, _ = buf
        for t in range(NXFER):
            pltpu.make_async_copy(
                tab_hbm.at[idx_v.at[pl.ds(t * XFER, XFER)]],
                rows_v.at[pl.ds(t * XFER, XFER)],
                gsem,
            ).wait()

    def wait_write(buf):
        _, rows_v, _, wsem = buf
        pltpu.make_async_copy(rows_v, out_hbm.at[pl.ds(0, GROUP)], wsem).wait()

    def process(gg, buf):
        idx_v, rows_v, gsem, wsem = buf
        wait_gathers(buf)

        def add_body(s, inner):
            for c in range(EMBED // LANES):
                p = pos_v[s, pl.ds(c * LANES, LANES)]
                for rep in range(REPS):
                    r = rep * SEQ + s
                    rows_v[r, pl.ds(c * LANES, LANES)] = (
                        rows_v[r, pl.ds(c * LANES, LANES)] + p
                    )
            return inner

        lax.fori_loop(0, SEQ, add_body, 0, unroll=8)
        off = base + gg * GROUP
        pltpu.async_copy(rows_v, out_hbm.at[pl.ds(off, GROUP)], wsem)

    issue(0, bufs[0])

    def loop_body(i, carry):
        g0 = i * 2

        @pl.when(i > 0)
        def _():
            wait_write(bufs[1])

        issue(g0 + 1, bufs[1])
        process(g0, bufs[0])
        process(g0 + 1, bufs[1])

        @pl.when(g0 + 2 < NGROUP)
        def _():
            wait_write(bufs[0])
            issue(g0 + 2, bufs[0])

        return carry

    lax.fori_loop(0, NGROUP // 2, loop_body, 0)
    wait_write(bufs[0])
    wait_write(bufs[1])


def kernel(x, token_embed, pos_embed):
    x_flat = x.reshape(FLAT).astype(jnp.int32)
    pos2d = pos_embed[0, :SEQ, :]
    out = _embed_kernel(x_flat, token_embed, pos2d)
    return out.reshape(BATCH, SEQ, EMBED)
